# baseline (speedup 1.0000x reference)
import functools

import jax
import jax.numpy as jnp
from jax import lax
from jax.experimental import pallas as pl
from jax.experimental.pallas import tpu as pltpu
from jax.experimental.pallas import tpu_sc as plsc

EMB = 64
LANES = 16
NW = 32
PANEL = 128
PER_CHUNK = 8
MCAP = 1040
CCAP = 64
CPAD = 80
OUT_PAD = 80


@functools.cache
def _sc_fused(batch, n_users_pad, n_items):
    n_panels = (n_users_pad + PANEL - 1) // PANEL
    max_j = (n_panels + NW - 1) // NW
    n_chunks = (max_j + PER_CHUNK - 1) // PER_CHUNK
    chunk_w = PER_CHUNK * PANEL
    n_ivecs = batch // LANES
    mesh = plsc.VectorSubcoreMesh(core_axis_name="c", subcore_axis_name="s")

    @functools.partial(
        pl.kernel,
        mesh=mesh,
        out_type=jax.ShapeDtypeStruct((batch + OUT_PAD,), jnp.float32),
        compiler_params=pltpu.CompilerParams(
            needs_layout_passes=False,
            use_tc_tiling_on_sc=True,
            disable_bounds_checks=True,
        ),
        scratch_types=[
            pltpu.VMEM((batch,), jnp.int32),
            pltpu.VMEM((batch,), jnp.int32),
            pltpu.VMEM((MCAP,), jnp.int32),
            pltpu.VMEM((MCAP,), jnp.int32),
            pltpu.VMEM((MCAP,), jnp.int32),
            pltpu.VMEM((MCAP,), jnp.int32),
            pltpu.VMEM((CPAD,), jnp.int32),
            pltpu.VMEM((CPAD,), jnp.int32),
            pltpu.VMEM((CPAD,), jnp.int32),
            pltpu.VMEM((CPAD,), jnp.float32),
            pltpu.VMEM((EMB, chunk_w), jnp.float32),
            pltpu.VMEM((CPAD, PANEL), jnp.float32),
            pltpu.VMEM((EMB * LANES,), jnp.float32),
            pltpu.VMEM((LANES,), jnp.float32),
            pltpu.SemaphoreType.DMA,
        ],
    )
    def sc_fn(user_hbm, item_hbm, utt_hbm, it_hbm, wb_hbm, b_hbm, out_hbm,
              u_v, v_v, m_cid, m_col, m_item, m_pos,
              c_col, c_item, c_pos, c_res, panels_v, itrows_v,
              wb_v, b_v, sem):
        wid = lax.axis_index("s") * 2 + lax.axis_index("c")
        iota = lax.iota(jnp.int32, LANES)

        pltpu.sync_copy(user_hbm, u_v)
        pltpu.sync_copy(item_hbm, v_v)
        pltpu.sync_copy(wb_hbm, wb_v)
        pltpu.sync_copy(b_hbm, b_v)
        bias = b_v[...]

        def scan_body(vi, cnt):
            u = u_v[pl.ds(vi * LANES, LANES)]
            v = v_v[pl.ds(vi * LANES, LANES)]
            m = ((u >> 7) & (NW - 1)) == wid
            cid = u >> 15
            col = ((u >> 12) & (PER_CHUNK - 1)) * PANEL + (u & (PANEL - 1))
            pos = vi * LANES + iota
            plsc.store_compressed(m_cid.at[pl.ds(cnt, LANES)], cid, mask=m)
            plsc.store_compressed(m_col.at[pl.ds(cnt, LANES)], col, mask=m)
            plsc.store_compressed(m_item.at[pl.ds(cnt, LANES)], v, mask=m)
            plsc.store_compressed(m_pos.at[pl.ds(cnt, LANES)], pos, mask=m)
            return cnt + jnp.max(plsc.all_reduce_population_count(m))

        n_members = lax.fori_loop(0, n_ivecs, scan_body, 0)
        n_mvecs = (n_members + LANES - 1) // LANES

        def chunk_body(ci, carry):
            for q in range(CPAD // LANES):
                c_col[pl.ds(q * LANES, LANES)] = jnp.zeros(
                    (LANES,), jnp.int32)
                c_item[pl.ds(q * LANES, LANES)] = jnp.zeros(
                    (LANES,), jnp.int32)
                c_pos[pl.ds(q * LANES, LANES)] = (
                    batch + q * LANES + iota)

            def cscan(vi, cnt):
                sl = pl.ds(vi * LANES, LANES)
                m = m_cid[sl] == ci
                plsc.store_compressed(c_col.at[pl.ds(cnt, LANES)],
                                      m_col[sl], mask=m)
                plsc.store_compressed(c_item.at[pl.ds(cnt, LANES)],
                                      m_item[sl], mask=m)
                plsc.store_compressed(c_pos.at[pl.ds(cnt, LANES)],
                                      m_pos[sl], mask=m)
                return cnt + jnp.max(plsc.all_reduce_population_count(m))

            m_chunk = lax.fori_loop(0, n_mvecs, cscan, 0)
            m_chunk = jnp.minimum(m_chunk, CCAP)

            @pl.when(m_chunk > 0)
            def _():
                descs = [pltpu.async_copy(it_hbm.at[c_item], itrows_v, sem)]
                for s in range(PER_CHUNK):
                    p = wid + NW * (ci * PER_CHUNK + s)
                    p = jnp.minimum(p, n_panels - 1)
                    descs.append(pltpu.async_copy(
                        utt_hbm.at[:, pl.ds(p * PANEL, PANEL)],
                        panels_v.at[:, pl.ds(s * PANEL, PANEL)], sem))
                for dd in descs:
                    dd.wait()

                def group_body(g, carry2):
                    off = pl.multiple_of(g * LANES, LANES)
                    ucol = c_col[pl.ds(off, LANES)]
                    slot = g * LANES + iota
                    acc = bias
                    for k in range(EMB):
                        ck = jnp.full((LANES,), k, dtype=jnp.int32)
                        gu = plsc.load_gather(panels_v, [ck, ucol])
                        gi = plsc.load_gather(itrows_v, [slot, ck])
                        acc = acc + gu * gi * wb_v[pl.ds(k * LANES, LANES)]
                    c_res[pl.ds(off, LANES)] = acc
                    return carry2

                n_groups = (m_chunk + LANES - 1) // LANES
                lax.fori_loop(0, n_groups, group_body, 0)
                pltpu.async_copy(c_res, out_hbm.at[c_pos], sem).wait()

            return carry

        lax.fori_loop(0, n_chunks, chunk_body, 0)

    return sc_fn


def kernel(user, item, user_table, item_table, W, b):
    batch = user.shape[0]
    n_users_pad = ((user_table.shape[0] + PANEL - 1) // PANEL) * PANEL
    utt = user_table.T
    it128 = jnp.pad(item_table, ((0, 0), (0, PANEL - EMB)))
    wb = jnp.broadcast_to(W.reshape(EMB, 1), (EMB, LANES)).reshape(-1)
    b16 = jnp.broadcast_to(b, (LANES,))
    fn = _sc_fused(batch, n_users_pad, item_table.shape[0])
    out = fn(user, item, utt, it128, wb, b16)
    return out[:batch].reshape(batch, 1)


# speedup vs baseline: 1.0027x; 1.0027x over previous
import functools

import jax
import jax.numpy as jnp
from jax import lax
from jax.experimental import pallas as pl
from jax.experimental.pallas import tpu as pltpu
from jax.experimental.pallas import tpu_sc as plsc

EMB = 64
LANES = 16
NW = 32
PANEL = 128
PER_CHUNK = 8
MCAP = 1040
CCAP = 64
CPAD = 80
OUT_PAD = 80


@functools.cache
def _sc_fused(batch, n_users_pad):
    n_panels = n_users_pad // PANEL
    per_tile = (n_panels + NW - 1) // NW
    n_chunks = (per_tile + PER_CHUNK - 1) // PER_CHUNK
    chunk_w = PER_CHUNK * PANEL
    span = per_tile * PANEL
    n_ivecs = batch // LANES
    mesh = plsc.VectorSubcoreMesh(core_axis_name="c", subcore_axis_name="s")

    @functools.partial(
        pl.kernel,
        mesh=mesh,
        out_type=jax.ShapeDtypeStruct((batch + OUT_PAD,), jnp.float32),
        compiler_params=pltpu.CompilerParams(
            needs_layout_passes=False,
            use_tc_tiling_on_sc=True,
            disable_bounds_checks=True,
        ),
        scratch_types=[
            pltpu.VMEM((batch,), jnp.int32),
            pltpu.VMEM((batch,), jnp.int32),
            pltpu.VMEM((MCAP,), jnp.int32),
            pltpu.VMEM((MCAP,), jnp.int32),
            pltpu.VMEM((MCAP,), jnp.int32),
            pltpu.VMEM((MCAP,), jnp.int32),
            pltpu.VMEM((CPAD,), jnp.int32),
            pltpu.VMEM((CPAD,), jnp.int32),
            pltpu.VMEM((CPAD,), jnp.int32),
            pltpu.VMEM((CPAD,), jnp.float32),
            pltpu.VMEM((EMB, chunk_w), jnp.float32),
            pltpu.VMEM((CPAD, PANEL), jnp.float32),
            pltpu.VMEM((EMB * LANES,), jnp.float32),
            pltpu.VMEM((LANES,), jnp.float32),
            pltpu.SemaphoreType.DMA,
        ],
    )
    def sc_fn(user_hbm, item_hbm, utt_hbm, it_hbm, wb_hbm, b_hbm, out_hbm,
              u_v, v_v, m_cid, m_u, m_item, m_pos,
              c_col, c_item, c_pos, c_res, chunk_v, itrows_v,
              wb_v, b_v, sem):
        wid = lax.axis_index("s") * 2 + lax.axis_index("c")
        iota = lax.iota(jnp.int32, LANES)
        lo = wid * span

        pltpu.sync_copy(user_hbm, u_v)
        pltpu.sync_copy(item_hbm, v_v)
        pltpu.sync_copy(wb_hbm, wb_v)
        pltpu.sync_copy(b_hbm, b_v)
        bias = b_v[...]

        def scan_body(vi, cnt):
            u = u_v[pl.ds(vi * LANES, LANES)]
            v = v_v[pl.ds(vi * LANES, LANES)]
            m = (u >= lo) & (u < lo + span)
            cid = (u - lo) >> 10
            pos = vi * LANES + iota
            plsc.store_compressed(m_cid.at[pl.ds(cnt, LANES)], cid, mask=m)
            plsc.store_compressed(m_u.at[pl.ds(cnt, LANES)], u, mask=m)
            plsc.store_compressed(m_item.at[pl.ds(cnt, LANES)], v, mask=m)
            plsc.store_compressed(m_pos.at[pl.ds(cnt, LANES)], pos, mask=m)
            return cnt + jnp.max(plsc.all_reduce_population_count(m))

        n_members = lax.fori_loop(0, n_ivecs, scan_body, 0)
        n_mvecs = (n_members + LANES - 1) // LANES

        def chunk_body(ci, carry):
            base = jnp.minimum(lo + ci * chunk_w,
                               n_panels * PANEL - chunk_w)
            for q in range(CPAD // LANES):
                c_col[pl.ds(q * LANES, LANES)] = jnp.zeros(
                    (LANES,), jnp.int32)
                c_item[pl.ds(q * LANES, LANES)] = jnp.zeros(
                    (LANES,), jnp.int32)
                c_pos[pl.ds(q * LANES, LANES)] = (
                    batch + q * LANES + iota)

            def cscan(vi, cnt):
                sl = pl.ds(vi * LANES, LANES)
                m = m_cid[sl] == ci
                plsc.store_compressed(c_col.at[pl.ds(cnt, LANES)],
                                      m_u[sl] - base, mask=m)
                plsc.store_compressed(c_item.at[pl.ds(cnt, LANES)],
                                      m_item[sl], mask=m)
                plsc.store_compressed(c_pos.at[pl.ds(cnt, LANES)],
                                      m_pos[sl], mask=m)
                return cnt + jnp.max(plsc.all_reduce_population_count(m))

            m_chunk = lax.fori_loop(0, n_mvecs, cscan, 0)
            m_chunk = jnp.minimum(m_chunk, CCAP)

            @pl.when(m_chunk > 0)
            def _():
                descs = [pltpu.async_copy(it_hbm.at[c_item], itrows_v, sem)]
                for r in range(EMB // 8):
                    descs.append(pltpu.async_copy(
                        utt_hbm.at[pl.ds(r * 8, 8), pl.ds(base, chunk_w)],
                        chunk_v.at[pl.ds(r * 8, 8), :], sem))
                for dd in descs:
                    dd.wait()

                def group_body(g, carry2):
                    off = pl.multiple_of(g * LANES, LANES)
                    ucol = c_col[pl.ds(off, LANES)]
                    slot = g * LANES + iota
                    acc = bias
                    for k in range(EMB):
                        ck = jnp.full((LANES,), k, dtype=jnp.int32)
                        gu = plsc.load_gather(chunk_v, [ck, ucol])
                        gi = plsc.load_gather(itrows_v, [slot, ck])
                        acc = acc + gu * gi * wb_v[pl.ds(k * LANES, LANES)]
                    c_res[pl.ds(off, LANES)] = acc
                    return carry2

                n_groups = (m_chunk + LANES - 1) // LANES
                lax.fori_loop(0, n_groups, group_body, 0)
                pltpu.async_copy(c_res, out_hbm.at[c_pos], sem).wait()

            return carry

        lax.fori_loop(0, n_chunks, chunk_body, 0)

    return sc_fn


def kernel(user, item, user_table, item_table, W, b):
    batch = user.shape[0]
    n_users_pad = ((user_table.shape[0] + PANEL - 1) // PANEL) * PANEL
    utt = user_table.T
    it128 = jnp.pad(item_table, ((0, 0), (0, PANEL - EMB)))
    wb = jnp.broadcast_to(W.reshape(EMB, 1), (EMB, LANES)).reshape(-1)
    b16 = jnp.broadcast_to(b, (LANES,))
    fn = _sc_fused(batch, n_users_pad)
    out = fn(user, item, utt, it128, wb, b16)
    return out[:batch].reshape(batch, 1)


# speedup vs baseline: 14.9294x; 14.8897x over previous
import functools

import jax
import jax.numpy as jnp
from jax import lax
from jax.experimental import pallas as pl
from jax.experimental.pallas import tpu as pltpu
from jax.experimental.pallas import tpu_sc as plsc

EMB = 64
ROW = 128
LANES = 16
CHUNK = 128


@functools.cache
def _sc_embed_dot(b_per_w, batch):
    n_chunks = b_per_w // CHUNK
    n_groups = b_per_w // LANES
    mesh = plsc.VectorSubcoreMesh(core_axis_name="c", subcore_axis_name="s")

    @functools.partial(
        pl.kernel,
        mesh=mesh,
        out_type=jax.ShapeDtypeStruct((batch,), jnp.float32),
        compiler_params=pltpu.CompilerParams(needs_layout_passes=False,
                                             use_tc_tiling_on_sc=True),
        scratch_types=[
            pltpu.VMEM((n_chunks, CHUNK), jnp.int32),
            pltpu.VMEM((n_chunks, CHUNK), jnp.int32),
            pltpu.VMEM((CHUNK, ROW), jnp.float32),
            pltpu.VMEM((CHUNK, ROW), jnp.float32),
            pltpu.VMEM((LANES,), jnp.float32),
            pltpu.VMEM((EMB * LANES,), jnp.float32),
            pltpu.VMEM((b_per_w,), jnp.float32),
            pltpu.SemaphoreType.DMA,
        ],
    )
    def sc_fn(user_hbm, item_hbm, ut_hbm, it_hbm, wb_hbm, b_hbm, out_hbm,
              uidx_v, iidx_v, urows_v, irows_v, b_v, wb_v, out_v, sem):
        num_cores = 2
        wid = lax.axis_index("s") * num_cores + lax.axis_index("c")
        base = wid * b_per_w

        for j in range(n_chunks):
            off = base + j * CHUNK
            pltpu.sync_copy(user_hbm.at[pl.ds(off, CHUNK)], uidx_v.at[j])
            pltpu.sync_copy(item_hbm.at[pl.ds(off, CHUNK)], iidx_v.at[j])

        pltpu.sync_copy(wb_hbm, wb_v)
        pltpu.sync_copy(b_hbm, b_v)
        bias = b_v[...]
        lane_iota = lax.iota(jnp.int32, LANES)

        for j in range(n_chunks):
            du = pltpu.async_copy(ut_hbm.at[uidx_v.at[j]], urows_v, sem)
            di = pltpu.async_copy(it_hbm.at[iidx_v.at[j]], irows_v, sem)
            du.wait()
            di.wait()

            def group_body(g, carry):
                row_idx = g * LANES + lane_iota
                acc = bias
                for k in range(EMB):
                    ck = jnp.full((LANES,), k, dtype=jnp.int32)
                    gu = plsc.load_gather(urows_v, [row_idx, ck])
                    gv = plsc.load_gather(irows_v, [row_idx, ck])
                    acc = acc + gu * gv * wb_v[pl.ds(k * LANES, LANES)]
                out_v[pl.ds(pl.multiple_of(j * CHUNK + g * LANES, LANES),
                            LANES)] = acc
                return carry

            lax.fori_loop(0, CHUNK // LANES, group_body, 0)

        pltpu.sync_copy(out_v, out_hbm.at[pl.ds(base, b_per_w)])

    return sc_fn


def kernel(user, item, user_table, item_table, W, b):
    batch = user.shape[0]
    num_workers = 32
    b_per_w = batch // num_workers
    ut128 = jnp.pad(user_table, ((0, 0), (0, ROW - EMB)))
    it128 = jnp.pad(item_table, ((0, 0), (0, ROW - EMB)))
    wb = jnp.broadcast_to(W.reshape(EMB, 1), (EMB, LANES)).reshape(-1)
    b16 = jnp.broadcast_to(b, (LANES,))
    fn = _sc_embed_dot(b_per_w, batch)
    out = fn(user, item, ut128, it128, wb, b16)
    return out.reshape(batch, 1)
